# baseline (speedup 1.0000x reference)
import jax
import jax.numpy as jnp
from jax.experimental import pallas as pl
from jax.experimental.pallas import tpu as pltpu

_TN = 1024
_QS = 255.0
_EPS = 1e-5


def _coeffs(stats_ref, g_ref, be_ref, count):
    stats = jnp.sum(stats_ref[...], axis=0)
    m = stats[0:1] / count
    v = stats[1:2] / count - m * m
    sc = g_ref[...] * jax.lax.rsqrt(v + _EPS)
    sh = be_ref[...] - m * sc
    return sc, sh


def _conv1_kernel(e0_ref, e1_ref, e2_ref, e3_ref, y_ref, w_ref,
                  out_ref, stats_ref, eq_ref, ybf_ref):
    i = pl.program_id(1)
    qn = e0_ref.shape[2]

    @pl.when(i == 0)
    def _():
        ybf_ref[...] = y_ref[0].astype(jnp.bfloat16)

    acc = jnp.zeros((e0_ref.shape[1], ybf_ref.shape[1]), jnp.float32)
    for k, e_ref in enumerate((e0_ref, e1_ref, e2_ref, e3_ref)):
        ef = e_ref[0]
        eq_ref[0, :, k * qn:(k + 1) * qn] = (
            jnp.minimum(ef * _QS, _QS).astype(jnp.uint8))
        acc += jnp.dot(ef.astype(jnp.bfloat16), ybf_ref[k * qn:(k + 1) * qn],
                       preferred_element_type=jnp.float32)
    a = jnp.dot(acc, w_ref[...], preferred_element_type=jnp.float32)
    out_ref[0] = a
    s = jnp.sum(a, axis=0, keepdims=True)
    s2 = jnp.sum(a * a, axis=0, keepdims=True)
    tile_stats = jnp.concatenate([s, s2], axis=0)[None]

    @pl.when(i == 0)
    def _():
        stats_ref[...] = jnp.zeros_like(stats_ref)

    stats_ref[...] += tile_stats


def _conv2_kernel(eq0_ref, eq1_ref, a1_ref, st1_ref, g1_ref, be1_ref, w_ref,
                  out_ref, stats_ref, max1_ref, sum1_ref, hbf_ref):
    i = pl.program_id(1)
    hn = eq0_ref.shape[2]
    count = jnp.float32(st1_ref.shape[0] * a1_ref.shape[1])

    @pl.when(i == 0)
    def _():
        sc, sh = _coeffs(st1_ref, g1_ref, be1_ref, count)
        h1 = jnp.maximum(a1_ref[0] * sc + sh, 0.0)
        hbf_ref[...] = h1.astype(jnp.bfloat16)
        max1_ref[0] = jnp.max(h1, axis=0, keepdims=True)
        sum1_ref[0] = jnp.sum(h1, axis=0, keepdims=True)

    acc = jnp.dot(eq0_ref[0].astype(jnp.bfloat16), hbf_ref[:hn],
                  preferred_element_type=jnp.float32)
    acc += jnp.dot(eq1_ref[0].astype(jnp.bfloat16), hbf_ref[hn:],
                   preferred_element_type=jnp.float32)
    acc = acc * jnp.float32(1.0 / _QS)
    a = jnp.dot(acc, w_ref[...], preferred_element_type=jnp.float32)
    out_ref[0] = a
    s = jnp.sum(a, axis=0, keepdims=True)
    s2 = jnp.sum(a * a, axis=0, keepdims=True)
    tile_stats = jnp.concatenate([s, s2], axis=0)[None]

    @pl.when(i == 0)
    def _():
        stats_ref[...] = jnp.zeros_like(stats_ref)

    stats_ref[...] += tile_stats


def _head_kernel(a2_ref, st2_ref, g2_ref, be2_ref, max1_ref, sum1_ref,
                 wm1_ref, bm1_ref, wm2_ref, bm2_ref, out_ref, gx_ref):
    b = pl.program_id(0)
    B = st2_ref.shape[0]
    N = a2_ref.shape[1]
    C = a2_ref.shape[2]
    count = jnp.float32(B * N)
    inv_n = jnp.float32(1.0 / N)

    sc, sh = _coeffs(st2_ref, g2_ref, be2_ref, count)
    h2 = jnp.maximum(a2_ref[0] * sc + sh, 0.0)
    row = jnp.concatenate(
        [max1_ref[b], sum1_ref[b] * inv_n,
         jnp.max(h2, axis=0, keepdims=True),
         jnp.sum(h2, axis=0, keepdims=True) * inv_n], axis=1)
    gx_ref[pl.ds(b, 1), :] = row

    @pl.when(b == B - 1)
    def _():
        hid = jnp.maximum(
            jnp.dot(gx_ref[...], wm1_ref[...],
                    preferred_element_type=jnp.float32) + bm1_ref[...], 0.0)
        out_ref[...] = (jnp.dot(hid, wm2_ref[...],
                                preferred_element_type=jnp.float32)
                        + bm2_ref[...])


_PAR = pltpu.CompilerParams(dimension_semantics=("parallel", "arbitrary"))


def kernel(x, edge, W1, b1, W2, b2, g1, be1, g2, be2, Wm1, bm1, Wm2, bm2):
    B, N, F = x.shape
    C1 = W1.shape[1]
    C2 = W2.shape[1]
    nc = Wm2.shape[1]
    TN = _TN
    nt = N // TN

    a1, stats1, eq = pl.pallas_call(
        _conv1_kernel,
        grid=(B, nt),
        in_specs=[
            pl.BlockSpec((1, TN, N // 4), lambda b, i: (b, i, 0)),
            pl.BlockSpec((1, TN, N // 4), lambda b, i: (b, i, 1)),
            pl.BlockSpec((1, TN, N // 4), lambda b, i: (b, i, 2)),
            pl.BlockSpec((1, TN, N // 4), lambda b, i: (b, i, 3)),
            pl.BlockSpec((1, N, F), lambda b, i: (b, 0, 0)),
            pl.BlockSpec((F, C1), lambda b, i: (0, 0)),
        ],
        out_specs=[
            pl.BlockSpec((1, TN, C1), lambda b, i: (b, i, 0)),
            pl.BlockSpec((1, 2, C1), lambda b, i: (b, 0, 0)),
            pl.BlockSpec((1, TN, N), lambda b, i: (b, i, 0)),
        ],
        out_shape=[
            jax.ShapeDtypeStruct((B, N, C1), jnp.float32),
            jax.ShapeDtypeStruct((B, 2, C1), jnp.float32),
            jax.ShapeDtypeStruct((B, N, N), jnp.uint8),
        ],
        scratch_shapes=[pltpu.VMEM((N, F), jnp.bfloat16)],
        compiler_params=_PAR,
    )(edge, edge, edge, edge, x, W1)

    a2, stats2, max1, sum1 = pl.pallas_call(
        _conv2_kernel,
        grid=(B, nt),
        in_specs=[
            pl.BlockSpec((1, TN, N // 2), lambda b, i: (b, i, 0)),
            pl.BlockSpec((1, TN, N // 2), lambda b, i: (b, i, 1)),
            pl.BlockSpec((1, N, C1), lambda b, i: (b, 0, 0)),
            pl.BlockSpec((B, 2, C1), lambda b, i: (0, 0, 0)),
            pl.BlockSpec((1, C1), lambda b, i: (0, 0)),
            pl.BlockSpec((1, C1), lambda b, i: (0, 0)),
            pl.BlockSpec((C1, C2), lambda b, i: (0, 0)),
        ],
        out_specs=[
            pl.BlockSpec((1, TN, C2), lambda b, i: (b, i, 0)),
            pl.BlockSpec((1, 2, C2), lambda b, i: (b, 0, 0)),
            pl.BlockSpec((1, 1, C1), lambda b, i: (b, 0, 0)),
            pl.BlockSpec((1, 1, C1), lambda b, i: (b, 0, 0)),
        ],
        out_shape=[
            jax.ShapeDtypeStruct((B, N, C2), jnp.float32),
            jax.ShapeDtypeStruct((B, 2, C2), jnp.float32),
            jax.ShapeDtypeStruct((B, 1, C1), jnp.float32),
            jax.ShapeDtypeStruct((B, 1, C1), jnp.float32),
        ],
        scratch_shapes=[pltpu.VMEM((N, C1), jnp.bfloat16)],
        compiler_params=_PAR,
    )(eq, eq, a1, stats1, g1.reshape(1, -1), be1.reshape(1, -1), W2)

    pred = pl.pallas_call(
        _head_kernel,
        grid=(B,),
        in_specs=[
            pl.BlockSpec((1, N, C2), lambda b: (b, 0, 0)),
            pl.BlockSpec((B, 2, C2), lambda b: (0, 0, 0)),
            pl.BlockSpec((1, C2), lambda b: (0, 0)),
            pl.BlockSpec((1, C2), lambda b: (0, 0)),
            pl.BlockSpec((B, 1, C1), lambda b: (0, 0, 0)),
            pl.BlockSpec((B, 1, C1), lambda b: (0, 0, 0)),
            pl.BlockSpec(Wm1.shape, lambda b: (0, 0)),
            pl.BlockSpec((1, Wm1.shape[1]), lambda b: (0, 0)),
            pl.BlockSpec(Wm2.shape, lambda b: (0, 0)),
            pl.BlockSpec((1, nc), lambda b: (0, 0)),
        ],
        out_specs=pl.BlockSpec((B, nc), lambda b: (0, 0)),
        out_shape=jax.ShapeDtypeStruct((B, nc), jnp.float32),
        scratch_shapes=[pltpu.VMEM((B, 4 * C2), jnp.float32)],
    )(a2, stats2, g2.reshape(1, -1), be2.reshape(1, -1), max1, sum1,
      Wm1, bm1.reshape(1, -1), Wm2, bm2.reshape(1, -1))

    return pred


# speedup vs baseline: 1.0860x; 1.0860x over previous
import jax
import jax.numpy as jnp
from jax.experimental import pallas as pl
from jax.experimental.pallas import tpu as pltpu

_TN = 2048
_QS = 255.0
_EPS = 1e-5


def _coeffs(stats_ref, g_ref, be_ref, count):
    stats = jnp.sum(stats_ref[...], axis=0)
    m = stats[0:1] / count
    v = stats[1:2] / count - m * m
    sc = g_ref[...] * jax.lax.rsqrt(v + _EPS)
    sh = be_ref[...] - m * sc
    return sc, sh


def _conv1_kernel(e0_ref, e1_ref, e2_ref, e3_ref, y_ref, w_ref,
                  out_ref, stats_ref, eq_ref, ybf_ref):
    i = pl.program_id(1)
    qn = e0_ref.shape[2]

    @pl.when(i == 0)
    def _():
        ybf_ref[...] = y_ref[0].astype(jnp.bfloat16)

    acc = jnp.zeros((e0_ref.shape[1], ybf_ref.shape[1]), jnp.float32)
    qs = jnp.bfloat16(_QS)
    for k, e_ref in enumerate((e0_ref, e1_ref, e2_ref, e3_ref)):
        eb = e_ref[0].astype(jnp.bfloat16)
        eq_ref[0, :, k * qn:(k + 1) * qn] = (
            jnp.minimum(eb * qs, qs).astype(jnp.uint8))
        acc += jnp.dot(eb, ybf_ref[k * qn:(k + 1) * qn],
                       preferred_element_type=jnp.float32)
    a = jnp.dot(acc, w_ref[...], preferred_element_type=jnp.float32)
    out_ref[0] = a
    s = jnp.sum(a, axis=0, keepdims=True)
    s2 = jnp.sum(a * a, axis=0, keepdims=True)
    tile_stats = jnp.concatenate([s, s2], axis=0)[None]

    @pl.when(i == 0)
    def _():
        stats_ref[...] = jnp.zeros_like(stats_ref)

    stats_ref[...] += tile_stats


def _conv2_kernel(eq0_ref, eq1_ref, a1_ref, st1_ref, g1_ref, be1_ref, w_ref,
                  out_ref, stats_ref, max1_ref, sum1_ref, hbf_ref):
    i = pl.program_id(1)
    hn = eq0_ref.shape[2]
    count = jnp.float32(st1_ref.shape[0] * a1_ref.shape[1])

    @pl.when(i == 0)
    def _():
        sc, sh = _coeffs(st1_ref, g1_ref, be1_ref, count)
        h1 = jnp.maximum(a1_ref[0] * sc + sh, 0.0)
        hbf_ref[...] = h1.astype(jnp.bfloat16)
        max1_ref[0] = jnp.max(h1, axis=0, keepdims=True)
        sum1_ref[0] = jnp.sum(h1, axis=0, keepdims=True)

    acc = jnp.dot(eq0_ref[0].astype(jnp.bfloat16), hbf_ref[:hn],
                  preferred_element_type=jnp.float32)
    acc += jnp.dot(eq1_ref[0].astype(jnp.bfloat16), hbf_ref[hn:],
                   preferred_element_type=jnp.float32)
    acc = acc * jnp.float32(1.0 / _QS)
    a = jnp.dot(acc, w_ref[...], preferred_element_type=jnp.float32)
    out_ref[0] = a
    s = jnp.sum(a, axis=0, keepdims=True)
    s2 = jnp.sum(a * a, axis=0, keepdims=True)
    tile_stats = jnp.concatenate([s, s2], axis=0)[None]

    @pl.when(i == 0)
    def _():
        stats_ref[...] = jnp.zeros_like(stats_ref)

    stats_ref[...] += tile_stats


def _head_kernel(a2_ref, st2_ref, g2_ref, be2_ref, max1_ref, sum1_ref,
                 wm1_ref, bm1_ref, wm2_ref, bm2_ref, out_ref, gx_ref):
    b = pl.program_id(0)
    B = st2_ref.shape[0]
    N = a2_ref.shape[1]
    C = a2_ref.shape[2]
    count = jnp.float32(B * N)
    inv_n = jnp.float32(1.0 / N)

    sc, sh = _coeffs(st2_ref, g2_ref, be2_ref, count)
    h2 = jnp.maximum(a2_ref[0] * sc + sh, 0.0)
    row = jnp.concatenate(
        [max1_ref[b], sum1_ref[b] * inv_n,
         jnp.max(h2, axis=0, keepdims=True),
         jnp.sum(h2, axis=0, keepdims=True) * inv_n], axis=1)
    gx_ref[pl.ds(b, 1), :] = row

    @pl.when(b == B - 1)
    def _():
        hid = jnp.maximum(
            jnp.dot(gx_ref[...], wm1_ref[...],
                    preferred_element_type=jnp.float32) + bm1_ref[...], 0.0)
        out_ref[...] = (jnp.dot(hid, wm2_ref[...],
                                preferred_element_type=jnp.float32)
                        + bm2_ref[...])


_PAR = pltpu.CompilerParams(dimension_semantics=("parallel", "arbitrary"))


def kernel(x, edge, W1, b1, W2, b2, g1, be1, g2, be2, Wm1, bm1, Wm2, bm2):
    B, N, F = x.shape
    C1 = W1.shape[1]
    C2 = W2.shape[1]
    nc = Wm2.shape[1]
    TN = _TN
    nt = N // TN

    a1, stats1, eq = pl.pallas_call(
        _conv1_kernel,
        grid=(B, nt),
        in_specs=[
            pl.BlockSpec((1, TN, N // 4), lambda b, i: (b, i, 0)),
            pl.BlockSpec((1, TN, N // 4), lambda b, i: (b, i, 1)),
            pl.BlockSpec((1, TN, N // 4), lambda b, i: (b, i, 2)),
            pl.BlockSpec((1, TN, N // 4), lambda b, i: (b, i, 3)),
            pl.BlockSpec((1, N, F), lambda b, i: (b, 0, 0)),
            pl.BlockSpec((F, C1), lambda b, i: (0, 0)),
        ],
        out_specs=[
            pl.BlockSpec((1, TN, C1), lambda b, i: (b, i, 0)),
            pl.BlockSpec((1, 2, C1), lambda b, i: (b, 0, 0)),
            pl.BlockSpec((1, TN, N), lambda b, i: (b, i, 0)),
        ],
        out_shape=[
            jax.ShapeDtypeStruct((B, N, C1), jnp.float32),
            jax.ShapeDtypeStruct((B, 2, C1), jnp.float32),
            jax.ShapeDtypeStruct((B, N, N), jnp.uint8),
        ],
        scratch_shapes=[pltpu.VMEM((N, F), jnp.bfloat16)],
        compiler_params=_PAR,
    )(edge, edge, edge, edge, x, W1)

    a2, stats2, max1, sum1 = pl.pallas_call(
        _conv2_kernel,
        grid=(B, nt),
        in_specs=[
            pl.BlockSpec((1, TN, N // 2), lambda b, i: (b, i, 0)),
            pl.BlockSpec((1, TN, N // 2), lambda b, i: (b, i, 1)),
            pl.BlockSpec((1, N, C1), lambda b, i: (b, 0, 0)),
            pl.BlockSpec((B, 2, C1), lambda b, i: (0, 0, 0)),
            pl.BlockSpec((1, C1), lambda b, i: (0, 0)),
            pl.BlockSpec((1, C1), lambda b, i: (0, 0)),
            pl.BlockSpec((C1, C2), lambda b, i: (0, 0)),
        ],
        out_specs=[
            pl.BlockSpec((1, TN, C2), lambda b, i: (b, i, 0)),
            pl.BlockSpec((1, 2, C2), lambda b, i: (b, 0, 0)),
            pl.BlockSpec((1, 1, C1), lambda b, i: (b, 0, 0)),
            pl.BlockSpec((1, 1, C1), lambda b, i: (b, 0, 0)),
        ],
        out_shape=[
            jax.ShapeDtypeStruct((B, N, C2), jnp.float32),
            jax.ShapeDtypeStruct((B, 2, C2), jnp.float32),
            jax.ShapeDtypeStruct((B, 1, C1), jnp.float32),
            jax.ShapeDtypeStruct((B, 1, C1), jnp.float32),
        ],
        scratch_shapes=[pltpu.VMEM((N, C1), jnp.bfloat16)],
        compiler_params=_PAR,
    )(eq, eq, a1, stats1, g1.reshape(1, -1), be1.reshape(1, -1), W2)

    pred = pl.pallas_call(
        _head_kernel,
        grid=(B,),
        in_specs=[
            pl.BlockSpec((1, N, C2), lambda b: (b, 0, 0)),
            pl.BlockSpec((B, 2, C2), lambda b: (0, 0, 0)),
            pl.BlockSpec((1, C2), lambda b: (0, 0)),
            pl.BlockSpec((1, C2), lambda b: (0, 0)),
            pl.BlockSpec((B, 1, C1), lambda b: (0, 0, 0)),
            pl.BlockSpec((B, 1, C1), lambda b: (0, 0, 0)),
            pl.BlockSpec(Wm1.shape, lambda b: (0, 0)),
            pl.BlockSpec((1, Wm1.shape[1]), lambda b: (0, 0)),
            pl.BlockSpec(Wm2.shape, lambda b: (0, 0)),
            pl.BlockSpec((1, nc), lambda b: (0, 0)),
        ],
        out_specs=pl.BlockSpec((B, nc), lambda b: (0, 0)),
        out_shape=jax.ShapeDtypeStruct((B, nc), jnp.float32),
        scratch_shapes=[pltpu.VMEM((B, 4 * C2), jnp.float32)],
    )(a2, stats2, g2.reshape(1, -1), be2.reshape(1, -1), max1, sum1,
      Wm1, bm1.reshape(1, -1), Wm2, bm2.reshape(1, -1))

    return pred


# speedup vs baseline: 1.1158x; 1.0274x over previous
import jax
import jax.numpy as jnp
from jax.experimental import pallas as pl
from jax.experimental.pallas import tpu as pltpu

_TN = 2048
_QS = 255.0
_EPS = 1e-5


def _coeffs(stats_ref, g_ref, be_ref, count):
    stats = jnp.sum(stats_ref[...], axis=0)
    m = stats[0:1] / count
    v = stats[1:2] / count - m * m
    sc = g_ref[...] * jax.lax.rsqrt(v + _EPS)
    sh = be_ref[...] - m * sc
    return sc, sh


def _conv1_kernel(e0, e1, e2, e3, e4, e5, e6, e7, y_ref, w_ref,
                  out_ref, stats_ref, eqa_ref, eqb_ref, ybf_ref):
    i = pl.program_id(1)
    es = (e0, e1, e2, e3, e4, e5, e6, e7)
    qn = e0.shape[2]

    @pl.when(i == 0)
    def _():
        ybf_ref[...] = y_ref[0].astype(jnp.bfloat16)

    acc = jnp.zeros((e0.shape[1], ybf_ref.shape[1]), jnp.float32)
    qs = jnp.bfloat16(_QS)
    for k, e_ref in enumerate(es):
        eb = e_ref[0].astype(jnp.bfloat16)
        q = jnp.minimum(eb * qs, qs).astype(jnp.uint8)
        eq_ref = eqa_ref if k < 4 else eqb_ref
        kk = k % 4
        eq_ref[0, :, kk * qn:(kk + 1) * qn] = q
        acc += jnp.dot(eb, ybf_ref[k * qn:(k + 1) * qn],
                       preferred_element_type=jnp.float32)
    a = jnp.dot(acc, w_ref[...], preferred_element_type=jnp.float32)
    out_ref[0] = a.astype(jnp.bfloat16)
    s = jnp.sum(a, axis=0, keepdims=True)
    s2 = jnp.sum(a * a, axis=0, keepdims=True)
    tile_stats = jnp.concatenate([s, s2], axis=0)[None]

    @pl.when(i == 0)
    def _():
        stats_ref[...] = jnp.zeros_like(stats_ref)

    stats_ref[...] += tile_stats


def _conv2_kernel(eqa_ref, eqb_ref, a1_ref, st1_ref, g1_ref, be1_ref, w_ref,
                  out_ref, stats_ref, max1_ref, sum1_ref, hbf_ref):
    i = pl.program_id(1)
    hn = eqa_ref.shape[2]
    count = jnp.float32(st1_ref.shape[0] * a1_ref.shape[1])

    @pl.when(i == 0)
    def _():
        sc, sh = _coeffs(st1_ref, g1_ref, be1_ref, count)
        h1 = jnp.maximum(a1_ref[0].astype(jnp.float32) * sc + sh, 0.0)
        hbf_ref[...] = h1.astype(jnp.bfloat16)
        max1_ref[0] = jnp.max(h1, axis=0, keepdims=True)
        sum1_ref[0] = jnp.sum(h1, axis=0, keepdims=True)

    acc = jnp.dot(eqa_ref[0].astype(jnp.bfloat16), hbf_ref[:hn],
                  preferred_element_type=jnp.float32)
    acc += jnp.dot(eqb_ref[0].astype(jnp.bfloat16), hbf_ref[hn:],
                   preferred_element_type=jnp.float32)
    acc = acc * jnp.float32(1.0 / _QS)
    a = jnp.dot(acc, w_ref[...], preferred_element_type=jnp.float32)
    out_ref[0] = a.astype(jnp.bfloat16)
    s = jnp.sum(a, axis=0, keepdims=True)
    s2 = jnp.sum(a * a, axis=0, keepdims=True)
    tile_stats = jnp.concatenate([s, s2], axis=0)[None]

    @pl.when(i == 0)
    def _():
        stats_ref[...] = jnp.zeros_like(stats_ref)

    stats_ref[...] += tile_stats


def _head_kernel(a2_ref, st2_ref, g2_ref, be2_ref, max1_ref, sum1_ref,
                 wm1_ref, bm1_ref, wm2_ref, bm2_ref, out_ref, gx_ref):
    b = pl.program_id(0)
    B = st2_ref.shape[0]
    N = a2_ref.shape[1]
    count = jnp.float32(B * N)
    inv_n = jnp.float32(1.0 / N)

    sc, sh = _coeffs(st2_ref, g2_ref, be2_ref, count)
    h2 = jnp.maximum(a2_ref[0].astype(jnp.float32) * sc + sh, 0.0)
    row = jnp.concatenate(
        [max1_ref[b], sum1_ref[b] * inv_n,
         jnp.max(h2, axis=0, keepdims=True),
         jnp.sum(h2, axis=0, keepdims=True) * inv_n], axis=1)
    gx_ref[pl.ds(b, 1), :] = row

    @pl.when(b == B - 1)
    def _():
        hid = jnp.maximum(
            jnp.dot(gx_ref[...], wm1_ref[...],
                    preferred_element_type=jnp.float32) + bm1_ref[...], 0.0)
        out_ref[...] = (jnp.dot(hid, wm2_ref[...],
                                preferred_element_type=jnp.float32)
                        + bm2_ref[...])


_PAR = pltpu.CompilerParams(dimension_semantics=("parallel", "arbitrary"))


def kernel(x, edge, W1, b1, W2, b2, g1, be1, g2, be2, Wm1, bm1, Wm2, bm2):
    B, N, F = x.shape
    C1 = W1.shape[1]
    C2 = W2.shape[1]
    nc = Wm2.shape[1]
    TN = _TN
    nt = N // TN

    e_spec = [pl.BlockSpec((1, TN, N // 8), (lambda k: (lambda b, i: (b, i, k)))(k))
              for k in range(8)]
    a1, stats1, eqa, eqb = pl.pallas_call(
        _conv1_kernel,
        grid=(B, nt),
        in_specs=e_spec + [
            pl.BlockSpec((1, N, F), lambda b, i: (b, 0, 0)),
            pl.BlockSpec((F, C1), lambda b, i: (0, 0)),
        ],
        out_specs=[
            pl.BlockSpec((1, TN, C1), lambda b, i: (b, i, 0)),
            pl.BlockSpec((1, 2, C1), lambda b, i: (b, 0, 0)),
            pl.BlockSpec((1, TN, N // 2), lambda b, i: (b, i, 0)),
            pl.BlockSpec((1, TN, N // 2), lambda b, i: (b, i, 0)),
        ],
        out_shape=[
            jax.ShapeDtypeStruct((B, N, C1), jnp.bfloat16),
            jax.ShapeDtypeStruct((B, 2, C1), jnp.float32),
            jax.ShapeDtypeStruct((B, N, N // 2), jnp.uint8),
            jax.ShapeDtypeStruct((B, N, N // 2), jnp.uint8),
        ],
        scratch_shapes=[pltpu.VMEM((N, F), jnp.bfloat16)],
        compiler_params=_PAR,
    )(*([edge] * 8), x, W1)

    a2, stats2, max1, sum1 = pl.pallas_call(
        _conv2_kernel,
        grid=(B, nt),
        in_specs=[
            pl.BlockSpec((1, TN, N // 2), lambda b, i: (b, i, 0)),
            pl.BlockSpec((1, TN, N // 2), lambda b, i: (b, i, 0)),
            pl.BlockSpec((1, N, C1), lambda b, i: (b, 0, 0)),
            pl.BlockSpec((B, 2, C1), lambda b, i: (0, 0, 0)),
            pl.BlockSpec((1, C1), lambda b, i: (0, 0)),
            pl.BlockSpec((1, C1), lambda b, i: (0, 0)),
            pl.BlockSpec((C1, C2), lambda b, i: (0, 0)),
        ],
        out_specs=[
            pl.BlockSpec((1, TN, C2), lambda b, i: (b, i, 0)),
            pl.BlockSpec((1, 2, C2), lambda b, i: (b, 0, 0)),
            pl.BlockSpec((1, 1, C1), lambda b, i: (b, 0, 0)),
            pl.BlockSpec((1, 1, C1), lambda b, i: (b, 0, 0)),
        ],
        out_shape=[
            jax.ShapeDtypeStruct((B, N, C2), jnp.bfloat16),
            jax.ShapeDtypeStruct((B, 2, C2), jnp.float32),
            jax.ShapeDtypeStruct((B, 1, C1), jnp.float32),
            jax.ShapeDtypeStruct((B, 1, C1), jnp.float32),
        ],
        scratch_shapes=[pltpu.VMEM((N, C1), jnp.bfloat16)],
        compiler_params=_PAR,
    )(eqa, eqb, a1, stats1, g1.reshape(1, -1), be1.reshape(1, -1), W2)

    pred = pl.pallas_call(
        _head_kernel,
        grid=(B,),
        in_specs=[
            pl.BlockSpec((1, N, C2), lambda b: (b, 0, 0)),
            pl.BlockSpec((B, 2, C2), lambda b: (0, 0, 0)),
            pl.BlockSpec((1, C2), lambda b: (0, 0)),
            pl.BlockSpec((1, C2), lambda b: (0, 0)),
            pl.BlockSpec((B, 1, C1), lambda b: (0, 0, 0)),
            pl.BlockSpec((B, 1, C1), lambda b: (0, 0, 0)),
            pl.BlockSpec(Wm1.shape, lambda b: (0, 0)),
            pl.BlockSpec((1, Wm1.shape[1]), lambda b: (0, 0)),
            pl.BlockSpec(Wm2.shape, lambda b: (0, 0)),
            pl.BlockSpec((1, nc), lambda b: (0, 0)),
        ],
        out_specs=pl.BlockSpec((B, nc), lambda b: (0, 0)),
        out_shape=jax.ShapeDtypeStruct((B, nc), jnp.float32),
        scratch_shapes=[pltpu.VMEM((B, 4 * C2), jnp.float32)],
    )(a2, stats2, g2.reshape(1, -1), be2.reshape(1, -1), max1, sum1,
      Wm1, bm1.reshape(1, -1), Wm2, bm2.reshape(1, -1))

    return pred
